# Initial kernel scaffold; baseline (speedup 1.0000x reference)
import jax
import jax.numpy as jnp
from jax.experimental import pallas as pl


def kernel(pose_sequence, conv1_w, conv1_b, conv2_w, conv2_b, pose_pool, mlp_w1, mlp_b1, mlp_w2, mlp_b2):
    raise NotImplementedError("write your pallas kernel here")


# speedup vs baseline: 13.3581x; 13.3581x over previous
import functools

import jax
import jax.numpy as jnp
from jax.experimental import pallas as pl
from jax.experimental.pallas import tpu as pltpu

POSE_DIM = 256
HIDDEN_DIM = 1024
EPI_OUT = 256
NUM_ANCHORS = 1024
TOPK = 5
BLK = 512
EPS = 1e-8


def _fused_kernel(xpad_ref, a1_ref, b1_ref, a2_ref, b2_ref,
                  pool_ref, pool_t_ref, w1t_ref, mb1_ref, w2t_ref, mb2_ref,
                  out_ref, *, seq_len):
    j = pl.program_id(1)
    s0 = j * BLK

    xh = xpad_ref[0, pl.ds(s0, BLK + 4), :]

    h = jnp.dot(xh[0:BLK + 2], a1_ref[0], preferred_element_type=jnp.float32)
    h += jnp.dot(xh[1:BLK + 3], a1_ref[1], preferred_element_type=jnp.float32)
    h += jnp.dot(xh[2:BLK + 4], a1_ref[2], preferred_element_type=jnp.float32)
    h = jax.nn.relu(h + b1_ref[0])

    pos = s0 - 1 + jax.lax.broadcasted_iota(jnp.int32, (BLK + 2, 1), 0)
    h = jnp.where((pos >= 0) & (pos < seq_len), h, 0.0)

    enc = jnp.dot(h[0:BLK], a2_ref[0], preferred_element_type=jnp.float32)
    enc += jnp.dot(h[1:BLK + 1], a2_ref[1], preferred_element_type=jnp.float32)
    enc += jnp.dot(h[2:BLK + 2], a2_ref[2], preferred_element_type=jnp.float32)
    enc += b2_ref[0]

    x = xh[2:BLK + 2]
    xnorm = jnp.sqrt(jnp.sum(x * x, axis=1, keepdims=True))
    xn = x / jnp.maximum(xnorm, EPS)
    pnorm = jnp.sqrt(jnp.sum(pool_t_ref[...] * pool_t_ref[...], axis=0,
                             keepdims=True))
    sim = jnp.dot(xn, pool_t_ref[...], preferred_element_type=jnp.float32)
    sim = sim / jnp.maximum(pnorm, EPS)

    neg = jnp.float32(-jnp.inf)
    cur = sim
    mask = jnp.zeros(sim.shape, dtype=jnp.bool_)
    row_max = jnp.max(cur, axis=1, keepdims=True)
    for _ in range(TOPK):
        m = jnp.max(cur, axis=1, keepdims=True)
        hit = cur == m
        mask = mask | hit
        cur = jnp.where(hit, neg, cur)

    ew = jnp.exp(sim - row_max)
    denom = jnp.sum(ew, axis=1, keepdims=True)
    w5 = jnp.where(mask, ew, 0.0)
    wp = jnp.dot(w5, pool_ref[...], preferred_element_type=jnp.float32)
    wp = wp / denom

    h1 = jnp.dot(x, w1t_ref[0:POSE_DIM], preferred_element_type=jnp.float32)
    h1 += jnp.dot(wp, w1t_ref[POSE_DIM:2 * POSE_DIM],
                  preferred_element_type=jnp.float32)
    h1 = jax.nn.relu(h1 + mb1_ref[0])
    out = jnp.dot(h1, w2t_ref[...], preferred_element_type=jnp.float32)
    out_ref[0, :, :] = out + mb2_ref[0] + enc


def kernel(pose_sequence, conv1_w, conv1_b, conv2_w, conv2_b,
           pose_pool, mlp_w1, mlp_b1, mlp_w2, mlp_b2):
    B, S, D = pose_sequence.shape

    xpad = jnp.pad(pose_sequence, ((0, 0), (2, 2), (0, 0)))
    a1 = jnp.transpose(conv1_w, (2, 1, 0))
    a2 = jnp.transpose(conv2_w, (2, 1, 0))
    pool_t = jnp.transpose(pose_pool, (1, 0))
    w1t = jnp.transpose(mlp_w1, (1, 0))
    w2t = jnp.transpose(mlp_w2, (1, 0))
    b1 = conv1_b.reshape(1, HIDDEN_DIM)
    b2 = conv2_b.reshape(1, EPI_OUT)
    mb1 = mlp_b1.reshape(1, HIDDEN_DIM)
    mb2 = mlp_b2.reshape(1, POSE_DIM)

    n_s = S // BLK
    grid = (B, n_s)

    full = lambda shape: pl.BlockSpec(shape, lambda b, j: (0,) * len(shape))

    return pl.pallas_call(
        functools.partial(_fused_kernel, seq_len=S),
        grid=grid,
        in_specs=[
            pl.BlockSpec((1, S + 4, D), lambda b, j: (b, 0, 0)),
            full((3, POSE_DIM, HIDDEN_DIM)),
            full((1, HIDDEN_DIM)),
            full((3, HIDDEN_DIM, EPI_OUT)),
            full((1, EPI_OUT)),
            full((NUM_ANCHORS, POSE_DIM)),
            full((POSE_DIM, NUM_ANCHORS)),
            full((2 * POSE_DIM, HIDDEN_DIM)),
            full((1, HIDDEN_DIM)),
            full((HIDDEN_DIM, POSE_DIM)),
            full((1, POSE_DIM)),
        ],
        out_specs=pl.BlockSpec((1, BLK, EPI_OUT), lambda b, j: (b, j, 0)),
        out_shape=jax.ShapeDtypeStruct((B, S, EPI_OUT), jnp.float32),
        compiler_params=pltpu.CompilerParams(
            dimension_semantics=("parallel", "arbitrary"),
        ),
    )(xpad, a1, b1, a2, b2, pose_pool, pool_t, w1t, mb1, w2t, mb2)


# speedup vs baseline: 15.8701x; 1.1881x over previous
import functools

import jax
import jax.numpy as jnp
from jax.experimental import pallas as pl
from jax.experimental.pallas import tpu as pltpu

POSE_DIM = 256
HIDDEN_DIM = 1024
EPI_OUT = 256
NUM_ANCHORS = 1024
TOPK = 5
BLK = 512
EPS = 1e-8


def _fused_kernel(xpad_ref, a1_ref, b1_ref, a2_ref, b2_ref,
                  pool_ref, pool_t_ref, w1t_ref, mb1_ref, w2t_ref, mb2_ref,
                  out_ref, *, seq_len):
    j = pl.program_id(1)
    s0 = j * BLK

    xh = xpad_ref[0, pl.ds(s0, BLK + 4), :]
    xh_b = xh.astype(jnp.bfloat16)

    h = jnp.dot(xh_b[0:BLK + 2], a1_ref[0], preferred_element_type=jnp.float32)
    h += jnp.dot(xh_b[1:BLK + 3], a1_ref[1], preferred_element_type=jnp.float32)
    h += jnp.dot(xh_b[2:BLK + 4], a1_ref[2], preferred_element_type=jnp.float32)
    h = jax.nn.relu(h + b1_ref[0])

    pos = s0 - 1 + jax.lax.broadcasted_iota(jnp.int32, (BLK + 2, 1), 0)
    h = jnp.where((pos >= 0) & (pos < seq_len), h, 0.0)
    h_b = h.astype(jnp.bfloat16)

    enc = jnp.dot(h_b[0:BLK], a2_ref[0], preferred_element_type=jnp.float32)
    enc += jnp.dot(h_b[1:BLK + 1], a2_ref[1], preferred_element_type=jnp.float32)
    enc += jnp.dot(h_b[2:BLK + 2], a2_ref[2], preferred_element_type=jnp.float32)
    enc += b2_ref[0]

    x = xh[2:BLK + 2]
    xnorm = jnp.sqrt(jnp.sum(x * x, axis=1, keepdims=True))
    xn = x / jnp.maximum(xnorm, EPS)
    pnorm = jnp.sqrt(jnp.sum(pool_t_ref[...] * pool_t_ref[...], axis=0,
                             keepdims=True))
    sim = jnp.dot(xn, pool_t_ref[...], preferred_element_type=jnp.float32)
    sim = sim / jnp.maximum(pnorm, EPS)

    neg = jnp.float32(-jnp.inf)
    cur = sim
    for _ in range(TOPK - 1):
        m = jnp.max(cur, axis=1, keepdims=True)
        cur = jnp.where(cur == m, neg, cur)
    t5 = jnp.max(cur, axis=1, keepdims=True)

    ew = jnp.exp(sim)
    denom = jnp.sum(ew, axis=1, keepdims=True)
    w5 = jnp.where(sim >= t5, ew, 0.0)
    wp = jnp.dot(w5, pool_ref[...], preferred_element_type=jnp.float32)
    wp = wp / denom

    h1 = jnp.dot(xh_b[2:BLK + 2], w1t_ref[0:POSE_DIM],
                 preferred_element_type=jnp.float32)
    h1 += jnp.dot(wp.astype(jnp.bfloat16), w1t_ref[POSE_DIM:2 * POSE_DIM],
                  preferred_element_type=jnp.float32)
    h1 = jax.nn.relu(h1 + mb1_ref[0])
    out = jnp.dot(h1.astype(jnp.bfloat16), w2t_ref[...],
                  preferred_element_type=jnp.float32)
    out_ref[0, :, :] = out + mb2_ref[0] + enc


def kernel(pose_sequence, conv1_w, conv1_b, conv2_w, conv2_b,
           pose_pool, mlp_w1, mlp_b1, mlp_w2, mlp_b2):
    B, S, D = pose_sequence.shape

    xpad = jnp.pad(pose_sequence, ((0, 0), (2, 2), (0, 0)))
    bf = jnp.bfloat16
    a1 = jnp.transpose(conv1_w, (2, 1, 0)).astype(bf)
    a2 = jnp.transpose(conv2_w, (2, 1, 0)).astype(bf)
    pool_t = jnp.transpose(pose_pool, (1, 0))
    w1t = jnp.transpose(mlp_w1, (1, 0)).astype(bf)
    w2t = jnp.transpose(mlp_w2, (1, 0)).astype(bf)
    b1 = conv1_b.reshape(1, HIDDEN_DIM)
    b2 = conv2_b.reshape(1, EPI_OUT)
    mb1 = mlp_b1.reshape(1, HIDDEN_DIM)
    mb2 = mlp_b2.reshape(1, POSE_DIM)

    n_s = S // BLK
    grid = (B, n_s)

    full = lambda shape: pl.BlockSpec(shape, lambda b, j: (0,) * len(shape))

    return pl.pallas_call(
        functools.partial(_fused_kernel, seq_len=S),
        grid=grid,
        in_specs=[
            pl.BlockSpec((1, S + 4, D), lambda b, j: (b, 0, 0)),
            full((3, POSE_DIM, HIDDEN_DIM)),
            full((1, HIDDEN_DIM)),
            full((3, HIDDEN_DIM, EPI_OUT)),
            full((1, EPI_OUT)),
            full((NUM_ANCHORS, POSE_DIM)),
            full((POSE_DIM, NUM_ANCHORS)),
            full((2 * POSE_DIM, HIDDEN_DIM)),
            full((1, HIDDEN_DIM)),
            full((HIDDEN_DIM, POSE_DIM)),
            full((1, POSE_DIM)),
        ],
        out_specs=pl.BlockSpec((1, BLK, EPI_OUT), lambda b, j: (b, j, 0)),
        out_shape=jax.ShapeDtypeStruct((B, S, EPI_OUT), jnp.float32),
        compiler_params=pltpu.CompilerParams(
            dimension_semantics=("parallel", "arbitrary"),
        ),
    )(xpad, a1, b1, a2, b2, pose_pool, pool_t, w1t, mb1, w2t, mb2)
